# baseline (speedup 1.0000x reference)
import functools

import jax
import jax.numpy as jnp
from jax.experimental import pallas as pl
from jax.experimental.pallas import tpu as pltpu

_K = 1024
_D = 64
_BETA = 0.25
_B = 1024


def _prep_body(emb_ref, ew2_ref, whilo_ref, e2_ref):
    ew = emb_ref[...]
    n = jnp.sqrt(jnp.sum(ew * ew, axis=1, keepdims=True))
    ewn = ew / jnp.maximum(n, 1e-12)
    ew2_ref[...] = (2.0 * ewn).astype(jnp.bfloat16)
    hi = ewn.astype(jnp.bfloat16)
    lo = (ewn - hi.astype(jnp.float32)).astype(jnp.bfloat16)
    whilo_ref[...] = jnp.concatenate([hi, lo], axis=1)
    e2_ref[...] = jnp.sum(ewn * ewn, axis=1, keepdims=True)


def _main_body(z_ref, ew2_ref, whilo_ref, e2_ref,
               zq_ref, idx_ref, counts_ref, loss_ref, counts_scr, loss_scr):
    i = pl.program_id(0)
    nsteps = pl.num_programs(0)

    @pl.when(i == 0)
    def _init():
        counts_scr[...] = jnp.zeros((_K, 1), jnp.float32)
        loss_scr[...] = jnp.zeros((1, 1), jnp.float32)

    z = jnp.swapaxes(z_ref[...], 0, 1)
    zn = z / jnp.maximum(jnp.sqrt(jnp.sum(z * z, axis=1, keepdims=True)), 1e-12)
    dot2 = jax.lax.dot_general(
        ew2_ref[...], zn.astype(jnp.bfloat16),
        (((1,), (1,)), ((), ())), preferred_element_type=jnp.float32)
    score = e2_ref[...] - dot2
    m = jnp.min(score, axis=0, keepdims=True)
    mask = score == m
    iota = jax.lax.broadcasted_iota(jnp.int32, score.shape, 0)
    idx = jnp.min(jnp.where(mask, iota, _K), axis=0, keepdims=True)
    idx_ref[...] = idx

    onehot = jnp.where(iota == idx, 1.0, 0.0).astype(jnp.bfloat16)
    zq2 = jax.lax.dot_general(
        whilo_ref[...], onehot,
        (((0,), (0,)), ((), ())), preferred_element_type=jnp.float32)
    zq_ref[...] = zq2[:_D, :] + zq2[_D:, :]

    counts_scr[...] += jnp.sum(jnp.where(mask, 1.0, 0.0), axis=1, keepdims=True)
    znorm2 = jnp.sum(zn * zn, axis=1, keepdims=True)
    loss_scr[...] += jnp.reshape(jnp.sum(znorm2) + jnp.sum(m), (1, 1))

    @pl.when(i == nsteps - 1)
    def _fini():
        counts_ref[...] = counts_scr[...]
        loss_ref[...] = loss_scr[...]


def _epi_body(total, counts_ref, loss_ref, cb_ref, vq_ref, perp_ref):
    cb = loss_ref[...] * (1.0 / (total * _D))
    p = counts_ref[...] * (1.0 / total)
    ent = -jnp.sum(p * jnp.log(p + 1e-10))
    cb_ref[...] = cb
    vq_ref[...] = cb + _BETA * cb
    perp_ref[...] = jnp.exp(ent) * jnp.ones((1, 1), jnp.float32)


def kernel(z_e, emb_weight):
    nbatch, ntok, _ = z_e.shape
    n_rows = nbatch * ntok
    z2d = jnp.transpose(z_e, (0, 2, 1)).reshape(nbatch * _D, ntok)

    ew2, whilo, e2 = pl.pallas_call(
        _prep_body,
        out_shape=[
            jax.ShapeDtypeStruct((_K, _D), jnp.bfloat16),
            jax.ShapeDtypeStruct((_K, 2 * _D), jnp.bfloat16),
            jax.ShapeDtypeStruct((_K, 1), jnp.float32),
        ],
    )(emb_weight)

    zq2d, idx, counts, loss = pl.pallas_call(
        _main_body,
        grid=(nbatch,),
        in_specs=[
            pl.BlockSpec((_D, _B), lambda i: (i, 0)),
            pl.BlockSpec((_K, _D), lambda i: (0, 0)),
            pl.BlockSpec((_K, 2 * _D), lambda i: (0, 0)),
            pl.BlockSpec((_K, 1), lambda i: (0, 0)),
        ],
        out_specs=[
            pl.BlockSpec((_D, _B), lambda i: (i, 0)),
            pl.BlockSpec((1, _B), lambda i: (0, i)),
            pl.BlockSpec((_K, 1), lambda i: (0, 0)),
            pl.BlockSpec((1, 1), lambda i: (0, 0)),
        ],
        out_shape=[
            jax.ShapeDtypeStruct((nbatch * _D, ntok), jnp.float32),
            jax.ShapeDtypeStruct((1, n_rows), jnp.int32),
            jax.ShapeDtypeStruct((_K, 1), jnp.float32),
            jax.ShapeDtypeStruct((1, 1), jnp.float32),
        ],
        scratch_shapes=[
            pltpu.VMEM((_K, 1), jnp.float32),
            pltpu.VMEM((1, 1), jnp.float32),
        ],
    )(z2d, ew2, whilo, e2)

    cb, vq, perp = pl.pallas_call(
        functools.partial(_epi_body, float(n_rows)),
        out_shape=[jax.ShapeDtypeStruct((1, 1), jnp.float32)] * 3,
    )(counts, loss)

    z_q = jnp.transpose(zq2d.reshape(nbatch, _D, ntok), (0, 2, 1))
    cb_s = cb[0, 0]
    return (z_q, vq[0, 0], cb_s, cb_s, perp[0, 0], idx.reshape(n_rows))


# speedup vs baseline: 1.0257x; 1.0257x over previous
import jax
import jax.numpy as jnp
from jax.experimental import pallas as pl
from jax.experimental.pallas import tpu as pltpu

_K = 1024
_D = 64
_BETA = 0.25
_B = 1024


def _body(z_ref, emb_ref, zq_ref, idx_ref, cb_ref, vq_ref, perp_ref,
          ew2_scr, whilo_scr, e2_scr, ones_scr, counts_scr, loss_scr):
    i = pl.program_id(0)
    nsteps = pl.num_programs(0)

    @pl.when(i == 0)
    def _init():
        ew = emb_ref[...]
        n = jnp.sqrt(jnp.sum(ew * ew, axis=1, keepdims=True))
        ewn = ew / jnp.maximum(n, 1e-12)
        ew2_scr[...] = (2.0 * ewn).astype(jnp.bfloat16)
        hi = ewn.astype(jnp.bfloat16)
        lo = (ewn - hi.astype(jnp.float32)).astype(jnp.bfloat16)
        whilo_scr[...] = jnp.concatenate([hi, lo], axis=1)
        e2_scr[...] = jnp.sum(ewn * ewn, axis=1, keepdims=True)
        ones_scr[...] = jnp.ones((_B, 1), jnp.bfloat16)
        counts_scr[...] = jnp.zeros((_K, 1), jnp.float32)
        loss_scr[...] = jnp.zeros((1, 1), jnp.float32)

    z = jnp.swapaxes(z_ref[...], 0, 1)
    zn = z / jnp.maximum(jnp.sqrt(jnp.sum(z * z, axis=1, keepdims=True)), 1e-12)
    dot2 = jax.lax.dot_general(
        ew2_scr[...], zn.astype(jnp.bfloat16),
        (((1,), (1,)), ((), ())), preferred_element_type=jnp.float32)
    score = e2_scr[...] - dot2
    m = jnp.min(score, axis=0, keepdims=True)
    mask = score == m
    iota = jax.lax.broadcasted_iota(jnp.int32, score.shape, 0)
    idx = jnp.min(jnp.where(mask, iota, _K), axis=0, keepdims=True)
    idx_ref[...] = idx

    onehot = jnp.where(iota == idx, 1.0, 0.0).astype(jnp.bfloat16)
    zq2 = jax.lax.dot_general(
        whilo_scr[...], onehot,
        (((0,), (0,)), ((), ())), preferred_element_type=jnp.float32)
    zq_ref[...] = zq2[:_D, :] + zq2[_D:, :]

    counts_scr[...] += jax.lax.dot_general(
        onehot, ones_scr[...],
        (((1,), (0,)), ((), ())), preferred_element_type=jnp.float32)
    znorm2 = jnp.sum(zn * zn, axis=1, keepdims=True)
    loss_scr[...] += jnp.reshape(jnp.sum(znorm2) + jnp.sum(m), (1, 1))

    @pl.when(i == nsteps - 1)
    def _fini():
        total = nsteps * _B
        cb = loss_scr[...] * (1.0 / (total * _D))
        p = counts_scr[...] * (1.0 / total)
        ent = -jnp.sum(p * jnp.log(p + 1e-10))
        cb_ref[...] = cb
        vq_ref[...] = cb + _BETA * cb
        perp_ref[...] = jnp.exp(ent) * jnp.ones((1, 1), jnp.float32)


def kernel(z_e, emb_weight):
    nbatch, ntok, _ = z_e.shape
    n_rows = nbatch * ntok
    z2d = jnp.transpose(z_e, (0, 2, 1)).reshape(nbatch * _D, ntok)

    zq2d, idx, cb, vq, perp = pl.pallas_call(
        _body,
        grid=(nbatch,),
        in_specs=[
            pl.BlockSpec((_D, _B), lambda i: (i, 0)),
            pl.BlockSpec((_K, _D), lambda i: (0, 0)),
        ],
        out_specs=[
            pl.BlockSpec((_D, _B), lambda i: (i, 0)),
            pl.BlockSpec((1, _B), lambda i: (0, i)),
            pl.BlockSpec((1, 1), lambda i: (0, 0)),
            pl.BlockSpec((1, 1), lambda i: (0, 0)),
            pl.BlockSpec((1, 1), lambda i: (0, 0)),
        ],
        out_shape=[
            jax.ShapeDtypeStruct((nbatch * _D, ntok), jnp.float32),
            jax.ShapeDtypeStruct((1, n_rows), jnp.int32),
            jax.ShapeDtypeStruct((1, 1), jnp.float32),
            jax.ShapeDtypeStruct((1, 1), jnp.float32),
            jax.ShapeDtypeStruct((1, 1), jnp.float32),
        ],
        scratch_shapes=[
            pltpu.VMEM((_K, _D), jnp.bfloat16),
            pltpu.VMEM((_K, 2 * _D), jnp.bfloat16),
            pltpu.VMEM((_K, 1), jnp.float32),
            pltpu.VMEM((_B, 1), jnp.bfloat16),
            pltpu.VMEM((_K, 1), jnp.float32),
            pltpu.VMEM((1, 1), jnp.float32),
        ],
    )(z2d, emb_weight)

    z_q = jnp.transpose(zq2d.reshape(nbatch, _D, ntok), (0, 2, 1))
    cb_s = cb[0, 0]
    return (z_q, vq[0, 0], cb_s, cb_s, perp[0, 0], idx.reshape(n_rows))


# speedup vs baseline: 1.1936x; 1.1636x over previous
import jax
import jax.numpy as jnp
from jax.experimental import pallas as pl
from jax.experimental.pallas import tpu as pltpu

_K = 1024
_D = 64
_BETA = 0.25
_B = 1024


def _body(z_ref, emb_ref, zq_ref, idx_ref, cb_ref, vq_ref, perp_ref,
          ew2_scr, whilo_scr, e2_scr, counts_scr, loss_scr):
    i = pl.program_id(0)
    nsteps = pl.num_programs(0)

    @pl.when(i == 0)
    def _init():
        ew = emb_ref[...]
        n = jnp.sqrt(jnp.sum(ew * ew, axis=1, keepdims=True))
        ewn = ew / jnp.maximum(n, 1e-12)
        ew2_scr[...] = (2.0 * ewn).astype(jnp.bfloat16)
        hi = ewn.astype(jnp.bfloat16)
        lo = (ewn - hi.astype(jnp.float32)).astype(jnp.bfloat16)
        whilo_scr[...] = jnp.concatenate([hi, lo], axis=1)
        e2_scr[...] = jnp.sum(ewn * ewn, axis=1, keepdims=True)
        counts_scr[...] = jnp.zeros((_K, 1), jnp.float32)
        loss_scr[...] = jnp.zeros((1, 1), jnp.float32)

    zblk = z_ref[...]
    za = jnp.swapaxes(zblk[:_D, :], 0, 1)
    zb = jnp.swapaxes(zblk[_D:, :], 0, 1)
    zn = jnp.concatenate([za, zb], axis=0)
    zn = zn / jnp.maximum(jnp.sqrt(jnp.sum(zn * zn, axis=1, keepdims=True)), 1e-12)
    dot2 = jax.lax.dot_general(
        ew2_scr[...], zn.astype(jnp.bfloat16),
        (((1,), (1,)), ((), ())), preferred_element_type=jnp.float32)
    score = e2_scr[...] - dot2
    m = jnp.min(score, axis=0, keepdims=True)
    mask = score == m
    iota = jax.lax.broadcasted_iota(jnp.int32, score.shape, 0)
    idx = jnp.min(jnp.where(mask, iota, _K), axis=0, keepdims=True)
    idx_ref[...] = idx

    onehot = jnp.where(iota == idx, 1.0, 0.0).astype(jnp.bfloat16)
    zq2 = jax.lax.dot_general(
        whilo_scr[...], onehot,
        (((0,), (0,)), ((), ())), preferred_element_type=jnp.float32)
    zq = zq2[:_D, :] + zq2[_D:, :]
    zq_ref[...] = jnp.concatenate([zq[:, :_B], zq[:, _B:]], axis=0)

    counts_scr[...] += jnp.sum(onehot.astype(jnp.float32), axis=1, keepdims=True)
    znorm2 = jnp.sum(zn * zn, axis=1, keepdims=True)
    loss_scr[...] += jnp.reshape(jnp.sum(znorm2) + jnp.sum(m), (1, 1))

    @pl.when(i == nsteps - 1)
    def _fini():
        total = nsteps * 2 * _B
        cb = loss_scr[...] * (1.0 / (total * _D))
        p = counts_scr[...] * (1.0 / total)
        ent = -jnp.sum(p * jnp.log(p + 1e-10))
        cb_ref[...] = cb
        vq_ref[...] = cb + _BETA * cb
        perp_ref[...] = jnp.exp(ent) * jnp.ones((1, 1), jnp.float32)


def kernel(z_e, emb_weight):
    nbatch, ntok, _ = z_e.shape
    n_rows = nbatch * ntok
    z2d = jnp.transpose(z_e, (0, 2, 1)).reshape(nbatch * _D, ntok)

    zq2d, idx, cb, vq, perp = pl.pallas_call(
        _body,
        grid=(nbatch // 2,),
        in_specs=[
            pl.BlockSpec((2 * _D, _B), lambda i: (i, 0)),
            pl.BlockSpec((_K, _D), lambda i: (0, 0)),
        ],
        out_specs=[
            pl.BlockSpec((2 * _D, _B), lambda i: (i, 0)),
            pl.BlockSpec((1, 2 * _B), lambda i: (0, i)),
            pl.BlockSpec((1, 1), lambda i: (0, 0)),
            pl.BlockSpec((1, 1), lambda i: (0, 0)),
            pl.BlockSpec((1, 1), lambda i: (0, 0)),
        ],
        out_shape=[
            jax.ShapeDtypeStruct((nbatch * _D, ntok), jnp.float32),
            jax.ShapeDtypeStruct((1, n_rows), jnp.int32),
            jax.ShapeDtypeStruct((1, 1), jnp.float32),
            jax.ShapeDtypeStruct((1, 1), jnp.float32),
            jax.ShapeDtypeStruct((1, 1), jnp.float32),
        ],
        scratch_shapes=[
            pltpu.VMEM((_K, _D), jnp.bfloat16),
            pltpu.VMEM((_K, 2 * _D), jnp.bfloat16),
            pltpu.VMEM((_K, 1), jnp.float32),
            pltpu.VMEM((_K, 1), jnp.float32),
            pltpu.VMEM((1, 1), jnp.float32),
        ],
    )(z2d, emb_weight)

    z_q = jnp.transpose(zq2d.reshape(nbatch, _D, ntok), (0, 2, 1))
    cb_s = cb[0, 0]
    return (z_q, vq[0, 0], cb_s, cb_s, perp[0, 0], idx.reshape(n_rows))
